# baseline (speedup 1.0000x reference)
import jax
import jax.numpy as jnp
from jax import lax
from jax.experimental import pallas as pl
from jax.experimental.pallas import tpu as pltpu
from jax.experimental.pallas import tpu_sc as plsc

VOCAB = 1000000
EMBED_DIM = 64
BATCH = 16384
HIST = 50

_NC = 2
_NS = 16
_NW = _NC * _NS

_N_ROWS = BATCH * HIST
_PER_W = _N_ROWS // _NW
_XR_W = BATCH // _NW
_GXR = 4
_GRP = _GXR * HIST
_NGRP = _PER_W // _GRP
_NBUF = 4


def kernel(x, table):
    mesh = plsc.VectorSubcoreMesh(core_axis_name="c", subcore_axis_name="s")

    scratch = [pltpu.VMEM((_XR_W, HIST), jnp.int32)]
    scratch += [pltpu.VMEM((_GXR, HIST, 2 * EMBED_DIM), jnp.float32)] * _NBUF
    scratch += [pltpu.SemaphoreType.DMA] * (2 * _NBUF)

    @pl.kernel(
        out_type=jax.ShapeDtypeStruct((BATCH, HIST, 2 * EMBED_DIM), jnp.float32),
        mesh=mesh,
        compiler_params=pltpu.CompilerParams(use_tc_tiling_on_sc=False),
        scratch_types=scratch,
    )
    def k(x_hbm, table_hbm, out_hbm, idx_v, *rest):
        bufs = rest[:_NBUF]
        gsem = rest[_NBUF:2 * _NBUF]
        wsem = rest[2 * _NBUF:]
        wid = lax.axis_index("s") * _NC + lax.axis_index("c")
        base = wid * _PER_W
        xbase = wid * _XR_W
        pltpu.sync_copy(x_hbm.at[pl.ds(xbase, _XR_W)], idx_v)

        def wait_gather(b):
            for i in range(_GXR):
                pltpu.make_async_copy(
                    table_hbm.at[idx_v.at[0]],
                    bufs[b].at[i], gsem[b]).wait()

        def wait_write(b):
            pltpu.make_async_copy(
                bufs[b], out_hbm.at[pl.ds(0, _GXR)], wsem[b]).wait()

        def fire_gather(b, j):
            for i in range(_GXR):
                pltpu.async_copy(
                    table_hbm.at[idx_v.at[j * _GXR + i]],
                    bufs[b].at[i], gsem[b])

        def fire_write(b, j):
            pltpu.async_copy(
                bufs[b], out_hbm.at[pl.ds(xbase + j * _GXR, _GXR)], wsem[b])

        for b in range(_NBUF):
            fire_gather(b, b)

        def outer(g, carry):
            for b in range(_NBUF):
                j = g * _NBUF + b
                wait_gather(b)
                fire_write(b, j)
                bp = (b - 1) % _NBUF
                jp = j - 1
                jn = jp + _NBUF

                @pl.when(jp >= 0)
                def _():
                    wait_write(bp)

                @pl.when(jnp.logical_and(jp >= 0, jn < _NGRP))
                def _():
                    fire_gather(bp, jn)
            return carry

        lax.fori_loop(0, _NGRP // _NBUF, outer, 0)
        for j in range(_NGRP - _NGRP % _NBUF, _NGRP):
            b = j % _NBUF
            wait_gather(b)
            fire_write(b, j)
            bp = (b - 1) % _NBUF
            wait_write(bp)
            jn = j - 1 + _NBUF
            if jn < _NGRP:
                fire_gather(bp, jn)
        wait_write((_NGRP - 1) % _NBUF)

    tpad = jnp.pad(table, ((0, 0), (0, EMBED_DIM)))
    out = k(x, tpad)
    return out[:, :, :EMBED_DIM]


# speedup vs baseline: 1.1421x; 1.1421x over previous
import jax
import jax.numpy as jnp
from jax import lax
from jax.experimental import pallas as pl
from jax.experimental.pallas import tpu as pltpu
from jax.experimental.pallas import tpu_sc as plsc

VOCAB = 1000000
EMBED_DIM = 64
BATCH = 16384
HIST = 50

_NC = 2
_NS = 16
_NW = _NC * _NS

_N_ROWS = BATCH * HIST
_PER_W = _N_ROWS // _NW
_XR_W = BATCH // _NW
_GXR = 8
_GRP = _GXR * HIST
_NGRP = _PER_W // _GRP
_NBUF = 4


def kernel(x, table):
    mesh = plsc.VectorSubcoreMesh(core_axis_name="c", subcore_axis_name="s")

    scratch = [pltpu.VMEM((_XR_W, HIST), jnp.int32)]
    scratch += [pltpu.VMEM((_GXR, HIST, EMBED_DIM), jnp.float32)] * _NBUF
    scratch += [pltpu.SemaphoreType.DMA] * (2 * _NBUF)

    @pl.kernel(
        out_type=jax.ShapeDtypeStruct((BATCH, HIST, EMBED_DIM), jnp.float32),
        mesh=mesh,
        compiler_params=pltpu.CompilerParams(use_tc_tiling_on_sc=False),
        scratch_types=scratch,
    )
    def k(x_hbm, table_hbm, out_hbm, idx_v, *rest):
        bufs = rest[:_NBUF]
        gsem = rest[_NBUF:2 * _NBUF]
        wsem = rest[2 * _NBUF:]
        wid = lax.axis_index("s") * _NC + lax.axis_index("c")
        base = wid * _PER_W
        xbase = wid * _XR_W
        pltpu.sync_copy(x_hbm.at[pl.ds(xbase, _XR_W)], idx_v)

        def wait_gather(b):
            for i in range(_GXR):
                pltpu.make_async_copy(
                    table_hbm.at[idx_v.at[0]],
                    bufs[b].at[i], gsem[b]).wait()

        def wait_write(b):
            pltpu.make_async_copy(
                bufs[b], out_hbm.at[pl.ds(0, _GXR)], wsem[b]).wait()

        def fire_gather(b, j):
            for i in range(_GXR):
                pltpu.async_copy(
                    table_hbm.at[idx_v.at[j * _GXR + i]],
                    bufs[b].at[i], gsem[b])

        def fire_write(b, j):
            pltpu.async_copy(
                bufs[b], out_hbm.at[pl.ds(xbase + j * _GXR, _GXR)], wsem[b])

        for b in range(_NBUF):
            fire_gather(b, b)

        def outer(g, carry):
            for b in range(_NBUF):
                j = g * _NBUF + b
                wait_gather(b)
                fire_write(b, j)
                bp = (b - 1) % _NBUF
                jp = j - 1
                jn = jp + _NBUF

                @pl.when(jp >= 0)
                def _():
                    wait_write(bp)

                @pl.when(jnp.logical_and(jp >= 0, jn < _NGRP))
                def _():
                    fire_gather(bp, jn)
            return carry

        lax.fori_loop(0, _NGRP // _NBUF, outer, 0)
        for j in range(_NGRP - _NGRP % _NBUF, _NGRP):
            b = j % _NBUF
            wait_gather(b)
            fire_write(b, j)
            bp = (b - 1) % _NBUF
            wait_write(bp)
            jn = j - 1 + _NBUF
            if jn < _NGRP:
                fire_gather(bp, jn)
        wait_write((_NGRP - 1) % _NBUF)

    return k(x, table)


# speedup vs baseline: 1.1932x; 1.0448x over previous
import jax
import jax.numpy as jnp
from jax import lax
from jax.experimental import pallas as pl
from jax.experimental.pallas import tpu as pltpu
from jax.experimental.pallas import tpu_sc as plsc

VOCAB = 1000000
EMBED_DIM = 64
BATCH = 16384
HIST = 50

_NC = 2
_NS = 16
_NW = _NC * _NS

_B_W = BATCH // _NW
_GRP = 256
_SPLIT = _B_W // _GRP
_NGRP = HIST * _SPLIT
_NBUF = 4


def kernel(x, table):
    mesh = plsc.VectorSubcoreMesh(core_axis_name="c", subcore_axis_name="s")

    scratch = [pltpu.VMEM((HIST, _B_W), jnp.int32)]
    scratch += [pltpu.VMEM((_GRP, EMBED_DIM), jnp.float32)] * _NBUF
    scratch += [pltpu.SemaphoreType.DMA] * (2 * _NBUF)

    @pl.kernel(
        out_type=jax.ShapeDtypeStruct((HIST, BATCH, EMBED_DIM), jnp.float32),
        mesh=mesh,
        compiler_params=pltpu.CompilerParams(use_tc_tiling_on_sc=False),
        scratch_types=scratch,
    )
    def k(xt_hbm, table_hbm, out_hbm, idx_v, *rest):
        bufs = rest[:_NBUF]
        gsem = rest[_NBUF:2 * _NBUF]
        wsem = rest[2 * _NBUF:]
        wid = lax.axis_index("s") * _NC + lax.axis_index("c")
        bbase = wid * _B_W
        pltpu.sync_copy(xt_hbm.at[:, pl.ds(bbase, _B_W)], idx_v)

        def wait_gather(b):
            pltpu.make_async_copy(
                table_hbm.at[pl.ds(0, _GRP)], bufs[b], gsem[b]).wait()

        def wait_write(b):
            pltpu.make_async_copy(
                bufs[b], out_hbm.at[0, pl.ds(0, _GRP)], wsem[b]).wait()

        def fire_gather(b, j):
            h = j // _SPLIT
            s = j % _SPLIT
            pltpu.async_copy(
                table_hbm.at[idx_v.at[h, pl.ds(s * _GRP, _GRP)]], bufs[b],
                gsem[b])

        def fire_write(b, j):
            h = j // _SPLIT
            s = j % _SPLIT
            pltpu.async_copy(
                bufs[b],
                out_hbm.at[h, pl.ds(bbase + s * _GRP, _GRP)], wsem[b])

        for b in range(_NBUF):
            fire_gather(b, b)

        def outer(g, carry):
            for b in range(_NBUF):
                j = g * _NBUF + b
                wait_gather(b)
                fire_write(b, j)
                bp = (b - 1) % _NBUF
                jp = j - 1
                jn = jp + _NBUF

                @pl.when(jp >= 0)
                def _():
                    wait_write(bp)

                @pl.when(jnp.logical_and(jp >= 0, jn < _NGRP))
                def _():
                    fire_gather(bp, jn)
            return carry

        lax.fori_loop(0, _NGRP // _NBUF, outer, 0)
        wait_write((_NGRP - 1) % _NBUF)

    out_t = k(x.T, table)
    return out_t.transpose(1, 0, 2)
